# Initial kernel scaffold; baseline (speedup 1.0000x reference)
import jax
import jax.numpy as jnp
from jax.experimental import pallas as pl


def kernel(points, features, m1_w0, m1_b0, m1_w1, m1_b1, m1_w2, m1_b2, m2_w0, m2_b0, m2_w1, m2_b1, m2_w2, m2_b2, mm_w0, mm_b0, mm_w1, mm_b1, mm_w2, mm_b2):
    raise NotImplementedError("write your pallas kernel here")


# speedup vs baseline: 4.8364x; 4.8364x over previous
import functools

import jax
import jax.numpy as jnp
from jax import lax
from jax.experimental import pallas as pl
from jax.experimental.pallas import tpu as pltpu
from jax.experimental.pallas import tpu_sc as plsc

KNN = 32
DIL = 2
K64 = KNN * DIL


_R2 = 128


def _cmpx(k0, p0, k1, p1, asc):
    less = k0 < k1
    sel = less == asc
    nk0 = jnp.where(sel, k0, k1)
    nk1 = jnp.where(sel, k1, k0)
    np0 = jnp.where(sel, p0, p1)
    np1 = jnp.where(sel, p1, p0)
    return nk0, np0, nk1, np1


def _sort64_axis0(key, pay, asc_list):
    l, r = key.shape[1], key.shape[2]
    for k in (2, 4, 8, 16, 32, 64):
        j = k // 2
        while j >= 1:
            g = 64 // (2 * j)
            ks = key.reshape(g, 2, j, l, r)
            ps = pay.reshape(g, 2, j, l, r)
            if k == 64:
                asc = asc_list[None]
            else:
                giota = lax.broadcasted_iota(jnp.int32, (g, 1, 1, 1), 0)
                asc = (((giota * (2 * j)) & k) == 0) == asc_list[None]
            k0, p0, k1, p1 = _cmpx(ks[:, 0], ps[:, 0], ks[:, 1], ps[:, 1], asc)
            key = jnp.stack([k0, k1], axis=1).reshape(64, l, r)
            pay = jnp.stack([p0, p1], axis=1).reshape(64, l, r)
            j //= 2
    return key, pay


def _merge64_axis0(key, pay, asc_list):
    l, r = key.shape[1], key.shape[2]
    for j in (32, 16, 8, 4, 2, 1):
        g = 64 // (2 * j)
        ks = key.reshape(g, 2, j, l, r)
        ps = pay.reshape(g, 2, j, l, r)
        k0, p0, k1, p1 = _cmpx(ks[:, 0], ps[:, 0], ks[:, 1], ps[:, 1],
                               asc_list[None])
        key = jnp.stack([k0, k1], axis=1).reshape(64, l, r)
        pay = jnp.stack([p0, p1], axis=1).reshape(64, l, r)
    return key, pay


def _bf16_round(x):
    r = lax.bitcast_convert_type(x, jnp.int32)
    r = (r + 0x7FFF + ((r >> 16) & 1)) & ~0xFFFF
    return lax.bitcast_convert_type(r, jnp.float32)


def _topk_body(pts_row_ref, pts_all_ref, idx_ref):
    r = _R2
    pr16 = _bf16_round(pts_row_ref[0])
    xr = pr16[0:1, :]
    yr = pr16[1:2, :]
    zr = pr16[2:3, :]
    pa = pts_all_ref[0]
    pa16 = _bf16_round(pa)
    xc = pa16[:, 0:1]
    yc = pa16[:, 1:2]
    zc = pa16[:, 2:3]
    inner = xc * xr + yc * yr + zc * zr
    fr = pts_row_ref[0]
    sq_r = (fr[0:1, :] * fr[0:1, :] + fr[1:2, :] * fr[1:2, :]
            + fr[2:3, :] * fr[2:3, :])
    sq_c = (pa[:, 0:1] * pa[:, 0:1] + pa[:, 1:2] * pa[:, 1:2]
            + pa[:, 2:3] * pa[:, 2:3])
    dist = (sq_r - 2.0 * inner) + sq_c

    key = dist.reshape(64, 64, r)
    a_io = lax.broadcasted_iota(jnp.int32, (64, 64, r), 0)
    b_io = lax.broadcasted_iota(jnp.int32, (64, 64, r), 1)
    pay = a_io * 64 + b_io

    def alt(l):
        lio = lax.broadcasted_iota(jnp.int32, (1, l, 1), 1)
        return (lio % 2) == 0

    key, pay = _sort64_axis0(key, pay, alt(64))

    l = 64
    while l > 1:
        l2 = l // 2
        ks = key.reshape(64, l2, 2, r)
        ps = pay.reshape(64, l2, 2, r)
        kx, px = ks[:, :, 0], ps[:, :, 0]
        ky, py = ks[:, :, 1], ps[:, :, 1]
        less = kx < ky
        key = jnp.where(less, kx, ky)
        pay = jnp.where(less, px, py)
        asc = alt(l2) if l2 > 1 else jnp.ones((1, 1, 1), dtype=bool)
        key, pay = _merge64_axis0(key, pay, asc)
        l = l2

    idx_ref[0] = jnp.transpose(pay.reshape(64, r), (1, 0))


def _topk64(points):
    b, _, n = points.shape
    pts_t = jnp.transpose(points, (0, 2, 1))
    return pl.pallas_call(
        _topk_body,
        grid=(b, n // _R2),
        in_specs=[
            pl.BlockSpec((1, 3, _R2), lambda i, j: (i, 0, j)),
            pl.BlockSpec((1, n, 3), lambda i, j: (i, 0, 0)),
        ],
        out_specs=pl.BlockSpec((1, _R2, K64), lambda i, j: (i, j, 0)),
        out_shape=jax.ShapeDtypeStruct((b, n, K64), jnp.int32),
    )(points, pts_t)


def _node_mlp_body(f_ref, w0_ref, b0_ref, w1_ref, b1_ref, w2_ref, b2_ref,
                   m_ref):
    f = f_ref[0]
    a = lax.dot_general(f, w0_ref[...], (((0,), (1,)), ((), ())),
                        preferred_element_type=jnp.float32)
    a = jnp.maximum(a + b0_ref[...], 0.0)
    a = lax.dot_general(a, w1_ref[...], (((1,), (1,)), ((), ())),
                        preferred_element_type=jnp.float32)
    a = jnp.maximum(a + b1_ref[...], 0.0)
    a = lax.dot_general(a, w2_ref[...], (((1,), (1,)), ((), ())),
                        preferred_element_type=jnp.float32)
    m_ref[0] = a + b2_ref[...]


def _node_mlp(features, w0, b0, w1, b1, w2, b2):
    b, c, n = features.shape
    co = w2.shape[0]
    full = lambda a: pl.BlockSpec(a.shape, lambda i: (0,) * a.ndim)
    args = (w0, b0.reshape(1, -1), w1, b1.reshape(1, -1), w2,
            b2.reshape(1, -1))
    return pl.pallas_call(
        _node_mlp_body,
        grid=(b,),
        in_specs=[pl.BlockSpec((1, c, n), lambda i: (i, 0, 0))] +
                 [full(a) for a in args],
        out_specs=pl.BlockSpec((1, n, co), lambda i: (i, 0, 0)),
        out_shape=jax.ShapeDtypeStruct((b, n, co), jnp.float32),
    )(features, *args)


_NC = 2
_NS = 16
_NW = _NC * _NS
_CHUNK = 4


def _gather_max_sc(idx1, idx2, t1, t2):
    bn, c = t1.shape
    pw = bn // _NW
    nchunks = pw // _CHUNK
    rows_per_chunk = _CHUNK * KNN
    idx_rows_pw = pw * KNN // 128

    mesh = plsc.VectorSubcoreMesh(core_axis_name="c", subcore_axis_name="s")

    @functools.partial(
        pl.kernel,
        mesh=mesh,
        out_type=[jax.ShapeDtypeStruct((bn, c), jnp.float32),
                  jax.ShapeDtypeStruct((bn, c), jnp.float32)],
        scratch_types=[
            pltpu.VMEM((idx_rows_pw, 128), jnp.int32),
            pltpu.VMEM((rows_per_chunk, c), jnp.float32),
            pltpu.VMEM((pw, c), jnp.float32),
            pltpu.SemaphoreType.DMA,
        ],
    )
    def kern(idx1_hbm, idx2_hbm, t1_hbm, t2_hbm, l1_hbm, l2_hbm,
             idx_v, rows_v, out_v, sem):
        w = lax.axis_index("s") * _NC + lax.axis_index("c")

        for idx_hbm, t_hbm, l_hbm in ((idx1_hbm, t1_hbm, l1_hbm),
                                      (idx2_hbm, t2_hbm, l2_hbm)):
            pltpu.sync_copy(idx_hbm.at[pl.ds(w * idx_rows_pw, idx_rows_pw)],
                            idx_v)

            def chunk_body(ci, _, t_hbm=t_hbm):
                pltpu.async_copy(t_hbm.at[idx_v.at[ci]], rows_v, sem).wait()
                for nloc in range(_CHUNK):
                    node = ci * _CHUNK + nloc
                    for j in range(c // 16):
                        acc = rows_v[nloc * KNN, pl.ds(j * 16, 16)]
                        for k in range(1, KNN):
                            acc = jnp.maximum(
                                acc, rows_v[nloc * KNN + k, pl.ds(j * 16, 16)])
                        out_v[node, pl.ds(j * 16, 16)] = acc
                return 0

            lax.fori_loop(0, nchunks, chunk_body, 0)
            pltpu.sync_copy(out_v, l_hbm.at[pl.ds(w * pw, pw)])

    return kern(idx1, idx2, t1, t2)


_NB = 1024


def _final_mlp_body(l1_ref, l2_ref, w0a_ref, w0b_ref, b0_ref, w1_ref, b1_ref,
                    w2_ref, b2_ref, out_ref):
    z = (lax.dot_general(l1_ref[0], w0a_ref[...], (((1,), (1,)), ((), ())),
                         preferred_element_type=jnp.float32) +
         lax.dot_general(l2_ref[0], w0b_ref[...], (((1,), (1,)), ((), ())),
                         preferred_element_type=jnp.float32))
    z = jnp.maximum(z + b0_ref[...], 0.0)
    z = lax.dot_general(z, w1_ref[...], (((1,), (1,)), ((), ())),
                        preferred_element_type=jnp.float32)
    z = jnp.maximum(z + b1_ref[...], 0.0)
    out = lax.dot_general(w2_ref[...], z, (((1,), (1,)), ((), ())),
                          preferred_element_type=jnp.float32)
    out_ref[0] = out + b2_ref[...]


def _final_mlp(l1, l2, w0, b0, w1, b1, w2, b2):
    b, n, c = l1.shape
    c3 = w2.shape[0]
    w0a = w0[:, :c]
    w0b = w0[:, c:]
    full = lambda a: pl.BlockSpec(a.shape, lambda i, j: (0,) * a.ndim)
    args = (w0a, w0b, b0.reshape(1, -1), w1, b1.reshape(1, -1), w2,
            b2.reshape(-1, 1))
    return pl.pallas_call(
        _final_mlp_body,
        grid=(b, n // _NB),
        in_specs=[pl.BlockSpec((1, _NB, c), lambda i, j: (i, j, 0)),
                  pl.BlockSpec((1, _NB, c), lambda i, j: (i, j, 0))] +
                 [full(a) for a in args],
        out_specs=pl.BlockSpec((1, c3, _NB), lambda i, j: (i, 0, j)),
        out_shape=jax.ShapeDtypeStruct((b, c3, n), jnp.float32),
    )(l1, l2, *args)


def kernel(points, features, m1_w0, m1_b0, m1_w1, m1_b1, m1_w2, m1_b2,
           m2_w0, m2_b0, m2_w1, m2_b1, m2_w2, m2_b2,
           mm_w0, mm_b0, mm_w1, mm_b1, mm_w2, mm_b2):
    b, c, n = features.shape
    bn = b * n

    idx64 = _topk64(points)

    m1 = _node_mlp(features, m1_w0, m1_b0, m1_w1, m1_b1, m1_w2, m1_b2)
    m2 = _node_mlp(features, m2_w0, m2_b0, m2_w1, m2_b1, m2_w2, m2_b2)

    offs = (jnp.arange(b, dtype=jnp.int32) * n)[:, None, None]
    idx1 = (idx64[:, :, :KNN] + offs).reshape(bn * KNN // 128, 128)
    idx2 = (idx64[:, :, ::DIL] + offs).reshape(bn * KNN // 128, 128)

    l1, l2 = _gather_max_sc(idx1, idx2,
                            m1.reshape(bn, c), m2.reshape(bn, c))

    return _final_mlp(l1.reshape(b, n, c), l2.reshape(b, n, c),
                      mm_w0, mm_b0, mm_w1, mm_b1, mm_w2, mm_b2)
